# baseline (speedup 1.0000x reference)
import jax
import jax.numpy as jnp
from jax import lax
from jax.experimental import pallas as pl
from jax.experimental.pallas import tpu as pltpu
from jax.experimental.pallas import tpu_sc as plsc

N = 100000
NP = 100352
BNN = NP // 16
E = 1600000
F_IN = 12
H = 32
HH = 16
G = 128

NEG = -3.4e38

EPAD = 1638400
ER = EPAD // 128
ROWS_B = ER // 32
ROWS_D = ER // 16
ER_REAL = E // 128
EA_ROWS = E * 3 // 128


def _a1_body(x_ref, w_ref, asv_ref, adv_ref, c_ref,
             h0_ref, h1_ref, asrc_ref, adst_ref, sl_ref):
    i = pl.program_id(0)
    bn = x_ref.shape[0]
    h = jnp.dot(x_ref[...], w_ref[...], preferred_element_type=jnp.float32)
    h0_ref[...] = h[:, :HH]
    h1_ref[...] = h[:, HH:]
    a_s = jnp.sum(h * asv_ref[...], axis=1)
    a_d = jnp.sum(h * adv_ref[...], axis=1)
    asrc_ref[pl.ds(i * bn, bn)] = a_s
    adst_ref[pl.ds(i * bn, bn)] = a_d
    t = a_s + a_d + c_ref[0, 0]
    sl_ref[pl.ds(i * bn, bn)] = jnp.where(t >= 0, t, 0.2 * t)


def _node_proj(x, W, att_src, att_dst, c):
    BN = BNN
    grid = NP // BN
    return pl.pallas_call(
        _a1_body,
        grid=(grid,),
        in_specs=[
            pl.BlockSpec((BN, F_IN), lambda i: (i, 0)),
            pl.BlockSpec((F_IN, H), lambda i: (0, 0)),
            pl.BlockSpec((1, H), lambda i: (0, 0)),
            pl.BlockSpec((1, H), lambda i: (0, 0)),
            pl.BlockSpec((1, 1), lambda i: (0, 0)),
        ],
        out_specs=[
            pl.BlockSpec((BN, HH), lambda i: (i, 0)),
            pl.BlockSpec((BN, HH), lambda i: (i, 0)),
            pl.BlockSpec((NP,), lambda i: (0,)),
            pl.BlockSpec((NP,), lambda i: (0,)),
            pl.BlockSpec((NP,), lambda i: (0,)),
        ],
        out_shape=[
            jax.ShapeDtypeStruct((NP, HH), jnp.float32),
            jax.ShapeDtypeStruct((NP, HH), jnp.float32),
            jax.ShapeDtypeStruct((NP,), jnp.float32),
            jax.ShapeDtypeStruct((NP,), jnp.float32),
            jax.ShapeDtypeStruct((NP,), jnp.float32),
        ],
    )(x, W, att_src.reshape(1, H), att_dst.reshape(1, H), c)


def _c_body(mp_ref, sl_ref, h0_ref, h1_ref,
            m_ref, den0_ref, a00_ref, a01_ref):
    i = pl.program_id(0)
    bn = h0_ref.shape[0]
    sli = sl_ref[pl.ds(i * bn, bn)]
    mp = jnp.max(mp_ref[:, pl.ds(i * bn, bn)], axis=0)
    m = jnp.maximum(mp, sli)
    selfex = jnp.exp(sli - m)
    m_ref[pl.ds(i * bn, bn)] = m
    den0_ref[pl.ds(i * bn, bn)] = selfex
    a00_ref[...] = h0_ref[...] * selfex[:, None]
    a01_ref[...] = h1_ref[...] * selfex[:, None]


def _combine_max(mpart, sl, h0, h1):
    BN = BNN
    grid = NP // BN
    return pl.pallas_call(
        _c_body,
        grid=(grid,),
        in_specs=[
            pl.BlockSpec((32, NP), lambda i: (0, 0)),
            pl.BlockSpec((NP,), lambda i: (0,)),
            pl.BlockSpec((BN, HH), lambda i: (i, 0)),
            pl.BlockSpec((BN, HH), lambda i: (i, 0)),
        ],
        out_specs=[
            pl.BlockSpec((NP,), lambda i: (0,)),
            pl.BlockSpec((NP,), lambda i: (0,)),
            pl.BlockSpec((BN, HH), lambda i: (i, 0)),
            pl.BlockSpec((BN, HH), lambda i: (i, 0)),
        ],
        out_shape=[
            jax.ShapeDtypeStruct((NP,), jnp.float32),
            jax.ShapeDtypeStruct((NP,), jnp.float32),
            jax.ShapeDtypeStruct((NP, HH), jnp.float32),
            jax.ShapeDtypeStruct((NP, HH), jnp.float32),
        ],
    )(mpart, sl, h0, h1)


def _out_block(n0_ref, n1_ref, den_ref, bias_ref, i, bn):
    acc = jnp.concatenate([n0_ref[...], n1_ref[...]], axis=1)
    den = den_ref[pl.ds(i * bn, bn)]
    outv = acc / (den[:, None] + 1e-16) + bias_ref[...]
    return jnp.maximum(outv, 0.0)


def _e1_body(n0_ref, n1_ref, den_ref, bias_ref, alw_ref, alb_ref, b_ref,
             gate_ref, mg_ref):
    i = pl.program_id(0)
    bn = n0_ref.shape[0]
    outv = _out_block(n0_ref, n1_ref, den_ref, bias_ref, i, bn)
    gate = jnp.sum(outv * alw_ref[...], axis=1) + alb_ref[0, 0]
    gate_ref[pl.ds(i * bn, bn)] = gate

    @pl.when(i == 0)
    def _():
        mg_ref[...] = jnp.full_like(mg_ref, NEG)

    iota = lax.broadcasted_iota(jnp.int32, (bn, G), 1)
    mask = iota == b_ref[pl.ds(i * bn, bn)][:, None]
    gv = jnp.where(mask, gate[:, None], NEG)
    mg_ref[...] = jnp.maximum(mg_ref[...], jnp.max(gv, axis=0, keepdims=True))


def _node_gate(num0, num1, den, bias, alw, alb, batch):
    BN = BNN
    grid = NP // BN
    return pl.pallas_call(
        _e1_body,
        grid=(grid,),
        in_specs=[
            pl.BlockSpec((BN, HH), lambda i: (i, 0)),
            pl.BlockSpec((BN, HH), lambda i: (i, 0)),
            pl.BlockSpec((NP,), lambda i: (0,)),
            pl.BlockSpec((1, H), lambda i: (0, 0)),
            pl.BlockSpec((1, H), lambda i: (0, 0)),
            pl.BlockSpec((1, 1), lambda i: (0, 0)),
            pl.BlockSpec((NP,), lambda i: (0,)),
        ],
        out_specs=[
            pl.BlockSpec((NP,), lambda i: (0,)),
            pl.BlockSpec((1, G), lambda i: (0, 0)),
        ],
        out_shape=[
            jax.ShapeDtypeStruct((NP,), jnp.float32),
            jax.ShapeDtypeStruct((1, G), jnp.float32),
        ],
    )(num0, num1, den, bias, alw, alb, batch)


def _e2_body(n0_ref, n1_ref, den_ref, bias_ref, gate_ref, b_ref, mg_ref,
             ow_ref, ob_ref, pn_ref, pd_ref, y_ref):
    i = pl.program_id(0)
    ng = pl.num_programs(0)
    bn = n0_ref.shape[0]

    @pl.when(i == 0)
    def _():
        pn_ref[...] = jnp.zeros_like(pn_ref)
        pd_ref[...] = jnp.zeros_like(pd_ref)

    outv = _out_block(n0_ref, n1_ref, den_ref, bias_ref, i, bn)
    gate = gate_ref[pl.ds(i * bn, bn)]
    mg = mg_ref[...]
    mgs = jnp.where(mg > -1e38, mg, 0.0)
    iota = lax.broadcasted_iota(jnp.int32, (bn, G), 1)
    bo = (iota == b_ref[pl.ds(i * bn, bn)][:, None]).astype(jnp.float32)
    mgb = jnp.sum(bo * mgs, axis=1)
    w = jnp.exp(gate - mgb)
    oww = outv * w[:, None]
    pn_ref[...] += lax.dot_general(bo, oww, (((0,), (0,)), ((), ())),
                                   preferred_element_type=jnp.float32)
    pd_ref[...] += lax.dot_general(bo, w[:, None], (((0,), (0,)), ((), ())),
                                   preferred_element_type=jnp.float32)

    @pl.when(i == ng - 1)
    def _():
        pool = pn_ref[...] / (pd_ref[...] + 1e-16)
        yv = jnp.dot(pool, ow_ref[...], preferred_element_type=jnp.float32)
        yv = yv + ob_ref[...]
        y_ref[...] = 1.0 / (1.0 + jnp.exp(-yv))


def _pool(num0, num1, den, bias, gate, batch, mg, out_W, out_b):
    BN = BNN
    grid = NP // BN
    return pl.pallas_call(
        _e2_body,
        grid=(grid,),
        in_specs=[
            pl.BlockSpec((BN, HH), lambda i: (i, 0)),
            pl.BlockSpec((BN, HH), lambda i: (i, 0)),
            pl.BlockSpec((NP,), lambda i: (0,)),
            pl.BlockSpec((1, H), lambda i: (0, 0)),
            pl.BlockSpec((NP,), lambda i: (0,)),
            pl.BlockSpec((NP,), lambda i: (0,)),
            pl.BlockSpec((1, G), lambda i: (0, 0)),
            pl.BlockSpec((H, 1), lambda i: (0, 0)),
            pl.BlockSpec((1, 1), lambda i: (0, 0)),
        ],
        out_specs=[
            pl.BlockSpec((G, H), lambda i: (0, 0)),
            pl.BlockSpec((G, 1), lambda i: (0, 0)),
            pl.BlockSpec((G, 1), lambda i: (0, 0)),
        ],
        out_shape=[
            jax.ShapeDtypeStruct((G, H), jnp.float32),
            jax.ShapeDtypeStruct((G, 1), jnp.float32),
            jax.ShapeDtypeStruct((G, 1), jnp.float32),
        ],
    )(num0, num1, den, bias, gate, batch, mg, out_W, out_b)


_MESH = plsc.VectorSubcoreMesh(core_axis_name="c", subcore_axis_name="s")
_SC_PARAMS = pltpu.CompilerParams(needs_layout_passes=False,
                                  use_tc_tiling_on_sc=False)


def _sc_pass1(src_hbm, dst_hbm, ea_hbm, v3_hbm, asrc_hbm, adst_hbm,
              l_hbm, mpart_hbm,
              asrc_sh, adst_sh, m_t, srcb, dstb, aeb, lb, avb, dvb, shb,
              v3b, s1, s2):
    c = lax.axis_index("c")
    s = lax.axis_index("s")
    wid = c * 16 + s

    @pl.when(s == 0)
    def _():
        pltpu.sync_copy(asrc_hbm, asrc_sh)
        pltpu.sync_copy(adst_hbm, adst_sh)

    pltpu.sync_copy(v3_hbm, v3b)
    vv3 = v3b[pl.ds(0, 16)]
    v30 = vv3[0]
    v31 = vv3[1]
    v32 = vv3[2]

    @pl.loop(0, NP // 16)
    def _(i):
        m_t[pl.ds(i * 16, 16)] = jnp.full((16,), NEG, jnp.float32)

    shb[pl.ds(0, 16)] = jnp.full((16,), -1.0, jnp.float32)
    shb[pl.ds(32, 16)] = jnp.full((16,), -2.0, jnp.float32)

    plsc.subcore_barrier()

    rowbase = wid * ROWS_B

    @pl.loop(0, ROWS_B // 8)
    def _(ch):
        rb = rowbase + ch * 8
        pltpu.sync_copy(src_hbm.at[pl.ds(rb, 8)], srcb)
        pltpu.sync_copy(dst_hbm.at[pl.ds(rb, 8)], dstb)
        earow = jnp.minimum(3 * rb, EA_ROWS - 24)
        pltpu.sync_copy(ea_hbm.at[pl.ds(earow, 24)], aeb)

        av_d = [pltpu.async_copy(asrc_sh.at[srcb.at[k]], avb.at[k], s1)
                for k in range(8)]
        dv_d = [pltpu.async_copy(adst_sh.at[dstb.at[k]], dvb.at[k], s2)
                for k in range(8)]
        for d in av_d:
            d.wait()
        for d in dv_d:
            d.wait()

        @pl.loop(0, 8)
        def _(k):
            padoff = jnp.where(rb + k < ER_REAL, 0.0, NEG)

            @pl.loop(0, 8)
            def _(j):
                sl16 = pl.ds(j * 16, 16)
                eidx = lax.iota(jnp.int32, 16) + (k * 128 + j * 16)
                f0 = eidx * 3
                e0 = plsc.load_gather(aeb, [f0 >> 7, f0 & 127])
                f1 = f0 + 1
                e1 = plsc.load_gather(aeb, [f1 >> 7, f1 & 127])
                f2 = f0 + 2
                e2 = plsc.load_gather(aeb, [f2 >> 7, f2 & 127])
                ae16 = e0 * v30 + e1 * v31 + e2 * v32
                a = avb[k, sl16] + dvb[k, sl16] + ae16
                lb[k, sl16] = jnp.where(a >= 0, a, 0.2 * a) + padoff

            @pl.loop(0, 8)
            def _(j):
                sl16 = pl.ds(j * 16, 16)
                kk, vv = plsc.sort_key_val(dstb[k, sl16], lb[k, sl16])
                kf = kk.astype(jnp.float32)
                shb[pl.ds(16, 16)] = kf
                shb[pl.ds(48, 16)] = vv
                for sh in (1, 2, 4, 8):
                    pk = shb[pl.ds(16 - sh, 16)]
                    pv = shb[pl.ds(48 - sh, 16)]
                    take = pk == kf
                    vv = jnp.where(take, jnp.maximum(vv, pv), vv)
                    if sh < 8:
                        shb[pl.ds(48, 16)] = vv
                nk = shb[pl.ds(17, 16)]
                islast = nk != kf
                cur = plsc.load_gather(m_t, [kk])
                plsc.store_scatter(m_t, [kk], jnp.maximum(cur, vv),
                                   mask=islast)

        pltpu.sync_copy(lb, l_hbm.at[pl.ds(rb, 8)])

    pltpu.sync_copy(m_t, mpart_hbm.at[wid])


def _run_pass1(srcp, dstp, ea2d, v3p, asrc1, adst1):
    kfn = pl.kernel(
        _sc_pass1,
        out_type=[
            jax.ShapeDtypeStruct((ER, 128), jnp.float32),
            jax.ShapeDtypeStruct((32, NP), jnp.float32),
        ],
        mesh=_MESH,
        scratch_types=[
            pltpu.MemorySpace.VMEM_SHARED((NP,), jnp.float32),
            pltpu.MemorySpace.VMEM_SHARED((NP,), jnp.float32),
            pltpu.MemorySpace.VMEM((NP,), jnp.float32),
            pltpu.MemorySpace.VMEM((8, 128), jnp.int32),
            pltpu.MemorySpace.VMEM((8, 128), jnp.int32),
            pltpu.MemorySpace.VMEM((24, 128), jnp.float32),
            pltpu.MemorySpace.VMEM((8, 128), jnp.float32),
            pltpu.MemorySpace.VMEM((8, 128), jnp.float32),
            pltpu.MemorySpace.VMEM((8, 128), jnp.float32),
            pltpu.MemorySpace.VMEM((64,), jnp.float32),
            pltpu.MemorySpace.VMEM((16,), jnp.float32),
            pltpu.SemaphoreType.DMA,
            pltpu.SemaphoreType.DMA,
        ],
        compiler_params=_SC_PARAMS,
    )
    return kfn(srcp, dstp, ea2d, v3p, asrc1, adst1)


def _sc_pass2(src_hbm, dst_hbm, l_hbm, m_hbm, h0_hbm, h1_hbm,
              a00_hbm, a01_hbm, den0_hbm,
              num0_hbm, num1_hbm, den_hbm,
              acc_sh, den_sh, m_sh, srcb, dstb, lbuf, mvb, exb, rbig,
              gs0, gs1, gs2, vsem, as0, as1, as2):
    gsems = (gs0, gs1, gs2)
    asems = (as0, as1, as2)
    c = lax.axis_index("c")
    s = lax.axis_index("s")
    rows0 = s * (NP // 16)
    rowbase = s * ROWS_D

    @pl.when(s == 0)
    def _():
        pltpu.sync_copy(m_hbm, m_sh)

    @pl.when(s == 1)
    def _():
        pltpu.sync_copy(den0_hbm, den_sh)

    @pl.when(c == 0)
    def _():
        pltpu.sync_copy(a00_hbm.at[pl.ds(rows0, NP // 16)],
                        acc_sh.at[pl.ds(rows0, NP // 16)])

    @pl.when(c == 1)
    def _():
        pltpu.sync_copy(a01_hbm.at[pl.ds(rows0, NP // 16)],
                        acc_sh.at[pl.ds(rows0, NP // 16)])

    plsc.subcore_barrier()

    def main_loop(h_hbm, with_den):
        @pl.loop(0, ROWS_D // 8)
        def _(ch):
            rb = rowbase + ch * 8
            pltpu.sync_copy(src_hbm.at[pl.ds(rb, 8)], srcb)
            pltpu.sync_copy(dst_hbm.at[pl.ds(rb, 8)], dstb)
            pltpu.sync_copy(l_hbm.at[pl.ds(rb, 8)], lbuf)

            mv_d = [pltpu.async_copy(m_sh.at[dstb.at[k]], mvb.at[k], vsem)
                    for k in range(8)]
            for d in mv_d:
                d.wait()
            g_d = [None] * 8
            for k in range(2):
                g_d[k] = pltpu.async_copy(h_hbm.at[srcb.at[k]],
                                          rbig.at[k % 3], gsems[k % 3])
            sc_d = [None] * 8
            for k in range(8):
                @pl.loop(0, 8)
                def _(j, k=k):
                    sl16 = pl.ds(j * 16, 16)
                    exb[k, sl16] = jnp.exp(lbuf[k, sl16] - mvb[k, sl16])

                g_d[k].wait()
                if k + 2 < 8:
                    if k >= 1:
                        sc_d[k - 1].wait()
                    g_d[k + 2] = pltpu.async_copy(
                        h_hbm.at[srcb.at[k + 2]], rbig.at[(k + 2) % 3],
                        gsems[(k + 2) % 3])

                @pl.loop(0, 8)
                def _(q, k=k):
                    exq = exb[k, pl.ds(q * 16, 16)]
                    for t in range(16):
                        jj = q * 16 + t
                        rbig[k % 3, jj, pl.ds(0, HH)] = (
                            rbig[k % 3, jj, pl.ds(0, HH)] * exq[t])

                sc_d[k] = pltpu.async_copy(rbig.at[k % 3],
                                           acc_sh.at[dstb.at[k]],
                                           asems[k % 3], add=True)

                if with_den:
                    pltpu.sync_copy(exb.at[k], den_sh.at[dstb.at[k]],
                                    add=True)

            sc_d[5].wait()
            sc_d[6].wait()
            sc_d[7].wait()

    @pl.when(c == 0)
    def _():
        main_loop(h0_hbm, True)

    @pl.when(c == 1)
    def _():
        main_loop(h1_hbm, False)

    plsc.subcore_barrier()

    @pl.when(c == 0)
    def _():
        pltpu.sync_copy(acc_sh.at[pl.ds(rows0, NP // 16)],
                        num0_hbm.at[pl.ds(rows0, NP // 16)])

    @pl.when(c == 1)
    def _():
        pltpu.sync_copy(acc_sh.at[pl.ds(rows0, NP // 16)],
                        num1_hbm.at[pl.ds(rows0, NP // 16)])

    @pl.when(jnp.logical_and(c == 0, s == 0))
    def _():
        pltpu.sync_copy(den_sh, den_hbm)


def _run_pass2(srcp, dstp, l2d, m1, h0, h1, a00, a01, den01):
    kfn = pl.kernel(
        _sc_pass2,
        out_type=[
            jax.ShapeDtypeStruct((NP, HH), jnp.float32),
            jax.ShapeDtypeStruct((NP, HH), jnp.float32),
            jax.ShapeDtypeStruct((NP,), jnp.float32),
        ],
        mesh=_MESH,
        scratch_types=[
            pltpu.MemorySpace.VMEM_SHARED((NP, HH), jnp.float32),
            pltpu.MemorySpace.VMEM_SHARED((NP,), jnp.float32),
            pltpu.MemorySpace.VMEM_SHARED((NP,), jnp.float32),
            pltpu.MemorySpace.VMEM((8, 128), jnp.int32),
            pltpu.MemorySpace.VMEM((8, 128), jnp.int32),
            pltpu.MemorySpace.VMEM((8, 128), jnp.float32),
            pltpu.MemorySpace.VMEM((8, 128), jnp.float32),
            pltpu.MemorySpace.VMEM((8, 128), jnp.float32),
            pltpu.MemorySpace.VMEM((3, 128, HH), jnp.float32),
            pltpu.SemaphoreType.DMA,
            pltpu.SemaphoreType.DMA,
            pltpu.SemaphoreType.DMA,
            pltpu.SemaphoreType.DMA,
            pltpu.SemaphoreType.DMA,
            pltpu.SemaphoreType.DMA,
            pltpu.SemaphoreType.DMA,
        ],
        compiler_params=_SC_PARAMS,
    )
    return kfn(srcp, dstp, l2d, m1, h0, h1, a00, a01, den01)


@jax.jit
def kernel(x, edge_index, edge_attr, batch, W, att_src, att_dst, W_edge,
           att_edge, bias_conv, att_lin_W, att_lin_b, out_W, out_b):
    v3 = W_edge @ att_edge
    eaf = edge_attr.reshape(E * 3)
    ea2d = eaf.reshape(EA_ROWS, 128)
    c = (jnp.mean(eaf[0::3]) * v3[0] + jnp.mean(eaf[1::3]) * v3[1]
         + jnp.mean(eaf[2::3]) * v3[2]).reshape(1, 1)
    v3p = jnp.pad(v3, (0, 13))

    xp = jnp.pad(x, ((0, NP - N), (0, 0)))
    batchp = jnp.pad(batch, (0, NP - N), constant_values=G)
    h0, h1, asrc, adst, sl = _node_proj(xp, W, att_src, att_dst, c)

    pad = EPAD - E
    srcp = jnp.pad(edge_index[0], (0, pad)).reshape(ER, 128)
    dstp = jnp.pad(edge_index[1], (0, pad)).reshape(ER, 128)

    l2d, mpart = _run_pass1(srcp, dstp, ea2d, v3p, asrc, adst)

    m, den0, a00, a01 = _combine_max(mpart, sl, h0, h1)

    num0, num1, den = _run_pass2(srcp, dstp, l2d, m, h0, h1, a00, a01, den0)

    bias2 = bias_conv.reshape(1, H)
    gate, mg = _node_gate(num0, num1, den, bias2,
                          att_lin_W.reshape(1, H),
                          att_lin_b.reshape(1, 1), batchp)

    _, _, y = _pool(num0, num1, den, bias2, gate, batchp, mg,
                    out_W, out_b.reshape(1, 1))
    return y


# speedup vs baseline: 1.0374x; 1.0374x over previous
import jax
import jax.numpy as jnp
from jax import lax
from jax.experimental import pallas as pl
from jax.experimental.pallas import tpu as pltpu
from jax.experimental.pallas import tpu_sc as plsc

N = 100000
NP = 100352
BNN = NP // 16
E = 1600000
F_IN = 12
H = 32
HH = 16
G = 128

NEG = -3.4e38

EPAD = 1638400
ER = EPAD // 128
ROWS_B = ER // 32
ROWS_D = ER // 16
ER_REAL = E // 128
EA_ROWS = E * 3 // 128


def _a1_body(x_ref, x8_ref, w_ref, w0_ref, w1_ref, asv_ref, adv_ref, c_ref,
             h0_ref, h1_ref, asrc_ref, adst_ref, sl_ref):
    i = pl.program_id(0)
    bn = x_ref.shape[0]
    h = jnp.dot(x_ref[...], w_ref[...], preferred_element_type=jnp.float32)
    h0_ref[...] = jnp.dot(x8_ref[...], w0_ref[...],
                          preferred_element_type=jnp.float32)
    h1_ref[...] = jnp.dot(x8_ref[...], w1_ref[...],
                          preferred_element_type=jnp.float32)
    a_s = jnp.sum(h * asv_ref[...], axis=1)
    a_d = jnp.sum(h * adv_ref[...], axis=1)
    asrc_ref[pl.ds(i * bn, bn)] = a_s
    adst_ref[pl.ds(i * bn, bn)] = a_d
    t = a_s + a_d + c_ref[0, 0]
    sl_ref[pl.ds(i * bn, bn)] = jnp.where(t >= 0, t, 0.2 * t)


def _node_proj(x, x8, W, W0big, W1big, att_src, att_dst, c):
    BN = BNN
    grid = NP // BN
    return pl.pallas_call(
        _a1_body,
        grid=(grid,),
        in_specs=[
            pl.BlockSpec((BN, F_IN), lambda i: (i, 0)),
            pl.BlockSpec((BN // 8, 8 * F_IN), lambda i: (i, 0)),
            pl.BlockSpec((F_IN, H), lambda i: (0, 0)),
            pl.BlockSpec((8 * F_IN, 128), lambda i: (0, 0)),
            pl.BlockSpec((8 * F_IN, 128), lambda i: (0, 0)),
            pl.BlockSpec((1, H), lambda i: (0, 0)),
            pl.BlockSpec((1, H), lambda i: (0, 0)),
            pl.BlockSpec((1, 1), lambda i: (0, 0)),
        ],
        out_specs=[
            pl.BlockSpec((BN * HH // 128, 128), lambda i: (i, 0)),
            pl.BlockSpec((BN * HH // 128, 128), lambda i: (i, 0)),
            pl.BlockSpec((NP,), lambda i: (0,)),
            pl.BlockSpec((NP,), lambda i: (0,)),
            pl.BlockSpec((NP,), lambda i: (0,)),
        ],
        out_shape=[
            jax.ShapeDtypeStruct((NP * HH // 128, 128), jnp.float32),
            jax.ShapeDtypeStruct((NP * HH // 128, 128), jnp.float32),
            jax.ShapeDtypeStruct((NP,), jnp.float32),
            jax.ShapeDtypeStruct((NP,), jnp.float32),
            jax.ShapeDtypeStruct((NP,), jnp.float32),
        ],
    )(x, x8, W, W0big, W1big, att_src.reshape(1, H),
      att_dst.reshape(1, H), c)


def _c_body(mp_ref, sl_ref, m_ref, den0_ref):
    i = pl.program_id(0)
    bn = BNN
    sli = sl_ref[pl.ds(i * bn, bn)]
    mp = jnp.max(mp_ref[:, pl.ds(i * bn, bn)], axis=0)
    m = jnp.maximum(mp, sli)
    selfex = jnp.exp(sli - m)
    m_ref[pl.ds(i * bn, bn)] = m
    den0_ref[pl.ds(i * bn, bn)] = selfex


def _c2_body(h0_ref, h1_ref, d8_ref, e8_ref, a00_ref, a01_ref):
    rep = jnp.dot(d8_ref[...], e8_ref[...],
                  preferred_element_type=jnp.float32)
    a00_ref[...] = h0_ref[...] * rep
    a01_ref[...] = h1_ref[...] * rep


def _scale_acc0(h0, h1, den0_2d, e816):
    BR = BNN * HH // 128
    grid = (NP * HH // 128) // BR
    return pl.pallas_call(
        _c2_body,
        grid=(grid,),
        in_specs=[
            pl.BlockSpec((BR, 128), lambda i: (i, 0)),
            pl.BlockSpec((BR, 128), lambda i: (i, 0)),
            pl.BlockSpec((BR, 8), lambda i: (i, 0)),
            pl.BlockSpec((8, 128), lambda i: (0, 0)),
        ],
        out_specs=[
            pl.BlockSpec((BR, 128), lambda i: (i, 0)),
            pl.BlockSpec((BR, 128), lambda i: (i, 0)),
        ],
        out_shape=[
            jax.ShapeDtypeStruct((NP * HH // 128, 128), jnp.float32),
            jax.ShapeDtypeStruct((NP * HH // 128, 128), jnp.float32),
        ],
    )(h0, h1, den0_2d, e816)


def _combine_max(mpart, sl):
    grid = NP // BNN
    return pl.pallas_call(
        _c_body,
        grid=(grid,),
        in_specs=[
            pl.BlockSpec((32, NP), lambda i: (0, 0)),
            pl.BlockSpec((NP,), lambda i: (0,)),
        ],
        out_specs=[
            pl.BlockSpec((NP,), lambda i: (0,)),
            pl.BlockSpec((NP,), lambda i: (0,)),
        ],
        out_shape=[
            jax.ShapeDtypeStruct((NP,), jnp.float32),
            jax.ShapeDtypeStruct((NP,), jnp.float32),
        ],
    )(mpart, sl)


def _out_block(n0_ref, n1_ref, den_ref, bias_ref, i, bn):
    acc = jnp.concatenate([n0_ref[...], n1_ref[...]], axis=1)
    den = den_ref[pl.ds(i * bn, bn)]
    outv = acc / (den[:, None] + 1e-16) + bias_ref[...]
    return jnp.maximum(outv, 0.0)


def _e1_body(n0_ref, n1_ref, den_ref, bias_ref, alw_ref, alb_ref, b_ref,
             gate_ref, mg_ref):
    i = pl.program_id(0)
    bn = n0_ref.shape[0]
    outv = _out_block(n0_ref, n1_ref, den_ref, bias_ref, i, bn)
    gate = jnp.sum(outv * alw_ref[...], axis=1) + alb_ref[0, 0]
    gate_ref[pl.ds(i * bn, bn)] = gate

    @pl.when(i == 0)
    def _():
        mg_ref[...] = jnp.full_like(mg_ref, NEG)

    iota = lax.broadcasted_iota(jnp.int32, (bn, G), 1)
    mask = iota == b_ref[pl.ds(i * bn, bn)][:, None]
    gv = jnp.where(mask, gate[:, None], NEG)
    mg_ref[...] = jnp.maximum(mg_ref[...], jnp.max(gv, axis=0, keepdims=True))


def _node_gate(num0, num1, den, bias, alw, alb, batch):
    BN = BNN
    grid = NP // BN
    return pl.pallas_call(
        _e1_body,
        grid=(grid,),
        in_specs=[
            pl.BlockSpec((BN, HH), lambda i: (i, 0)),
            pl.BlockSpec((BN, HH), lambda i: (i, 0)),
            pl.BlockSpec((NP,), lambda i: (0,)),
            pl.BlockSpec((1, H), lambda i: (0, 0)),
            pl.BlockSpec((1, H), lambda i: (0, 0)),
            pl.BlockSpec((1, 1), lambda i: (0, 0)),
            pl.BlockSpec((NP,), lambda i: (0,)),
        ],
        out_specs=[
            pl.BlockSpec((NP,), lambda i: (0,)),
            pl.BlockSpec((1, G), lambda i: (0, 0)),
        ],
        out_shape=[
            jax.ShapeDtypeStruct((NP,), jnp.float32),
            jax.ShapeDtypeStruct((1, G), jnp.float32),
        ],
    )(num0, num1, den, bias, alw, alb, batch)


def _e2_body(n0_ref, n1_ref, den_ref, bias_ref, gate_ref, b_ref, mg_ref,
             ow_ref, ob_ref, pn_ref, pd_ref, y_ref):
    i = pl.program_id(0)
    ng = pl.num_programs(0)
    bn = n0_ref.shape[0]

    @pl.when(i == 0)
    def _():
        pn_ref[...] = jnp.zeros_like(pn_ref)
        pd_ref[...] = jnp.zeros_like(pd_ref)

    outv = _out_block(n0_ref, n1_ref, den_ref, bias_ref, i, bn)
    gate = gate_ref[pl.ds(i * bn, bn)]
    mg = mg_ref[...]
    mgs = jnp.where(mg > -1e38, mg, 0.0)
    iota = lax.broadcasted_iota(jnp.int32, (bn, G), 1)
    bo = (iota == b_ref[pl.ds(i * bn, bn)][:, None]).astype(jnp.float32)
    mgb = jnp.sum(bo * mgs, axis=1)
    w = jnp.exp(gate - mgb)
    oww = outv * w[:, None]
    pn_ref[...] += lax.dot_general(bo, oww, (((0,), (0,)), ((), ())),
                                   preferred_element_type=jnp.float32)
    pd_ref[...] += lax.dot_general(bo, w[:, None], (((0,), (0,)), ((), ())),
                                   preferred_element_type=jnp.float32)

    @pl.when(i == ng - 1)
    def _():
        pool = pn_ref[...] / (pd_ref[...] + 1e-16)
        yv = jnp.dot(pool, ow_ref[...], preferred_element_type=jnp.float32)
        yv = yv + ob_ref[...]
        y_ref[...] = 1.0 / (1.0 + jnp.exp(-yv))


def _pool(num0, num1, den, bias, gate, batch, mg, out_W, out_b):
    BN = BNN
    grid = NP // BN
    return pl.pallas_call(
        _e2_body,
        grid=(grid,),
        in_specs=[
            pl.BlockSpec((BN, HH), lambda i: (i, 0)),
            pl.BlockSpec((BN, HH), lambda i: (i, 0)),
            pl.BlockSpec((NP,), lambda i: (0,)),
            pl.BlockSpec((1, H), lambda i: (0, 0)),
            pl.BlockSpec((NP,), lambda i: (0,)),
            pl.BlockSpec((NP,), lambda i: (0,)),
            pl.BlockSpec((1, G), lambda i: (0, 0)),
            pl.BlockSpec((H, 1), lambda i: (0, 0)),
            pl.BlockSpec((1, 1), lambda i: (0, 0)),
        ],
        out_specs=[
            pl.BlockSpec((G, H), lambda i: (0, 0)),
            pl.BlockSpec((G, 1), lambda i: (0, 0)),
            pl.BlockSpec((G, 1), lambda i: (0, 0)),
        ],
        out_shape=[
            jax.ShapeDtypeStruct((G, H), jnp.float32),
            jax.ShapeDtypeStruct((G, 1), jnp.float32),
            jax.ShapeDtypeStruct((G, 1), jnp.float32),
        ],
    )(num0, num1, den, bias, gate, batch, mg, out_W, out_b)


_MESH = plsc.VectorSubcoreMesh(core_axis_name="c", subcore_axis_name="s")
_SC_PARAMS = pltpu.CompilerParams(needs_layout_passes=False,
                                  use_tc_tiling_on_sc=False)


def _sc_pass1(src_hbm, dst_hbm, ea_hbm, v3_hbm, asrc_hbm, adst_hbm,
              l_hbm, mpart_hbm,
              asrc_sh, adst_sh, m_t, srcb, dstb, aeb, lb, avb, dvb, shb,
              v3b, s1, s2):
    c = lax.axis_index("c")
    s = lax.axis_index("s")
    wid = c * 16 + s

    @pl.when(s == 0)
    def _():
        pltpu.sync_copy(asrc_hbm, asrc_sh)
        pltpu.sync_copy(adst_hbm, adst_sh)

    pltpu.sync_copy(v3_hbm, v3b)
    vv3 = v3b[pl.ds(0, 16)]
    v30 = vv3[0]
    v31 = vv3[1]
    v32 = vv3[2]

    @pl.loop(0, NP // 16)
    def _(i):
        m_t[pl.ds(i * 16, 16)] = jnp.full((16,), NEG, jnp.float32)

    shb[pl.ds(0, 16)] = jnp.full((16,), -1.0, jnp.float32)
    shb[pl.ds(32, 16)] = jnp.full((16,), -2.0, jnp.float32)

    plsc.subcore_barrier()

    rowbase = wid * ROWS_B

    @pl.loop(0, ROWS_B // 8)
    def _(ch):
        rb = rowbase + ch * 8
        pltpu.sync_copy(src_hbm.at[pl.ds(rb, 8)], srcb)
        pltpu.sync_copy(dst_hbm.at[pl.ds(rb, 8)], dstb)
        earow = jnp.minimum(3 * rb, EA_ROWS - 24)
        pltpu.sync_copy(ea_hbm.at[pl.ds(earow, 24)], aeb)

        av_d = [pltpu.async_copy(asrc_sh.at[srcb.at[k]], avb.at[k], s1)
                for k in range(8)]
        dv_d = [pltpu.async_copy(adst_sh.at[dstb.at[k]], dvb.at[k], s2)
                for k in range(8)]
        for d in av_d:
            d.wait()
        for d in dv_d:
            d.wait()

        @pl.loop(0, 8)
        def _(k):
            padoff = jnp.where(rb + k < ER_REAL, 0.0, NEG)

            @pl.loop(0, 8)
            def _(j):
                sl16 = pl.ds(j * 16, 16)
                eidx = lax.iota(jnp.int32, 16) + (k * 128 + j * 16)
                f0 = eidx * 3
                e0 = plsc.load_gather(aeb, [f0 >> 7, f0 & 127])
                f1 = f0 + 1
                e1 = plsc.load_gather(aeb, [f1 >> 7, f1 & 127])
                f2 = f0 + 2
                e2 = plsc.load_gather(aeb, [f2 >> 7, f2 & 127])
                ae16 = e0 * v30 + e1 * v31 + e2 * v32
                a = avb[k, sl16] + dvb[k, sl16] + ae16
                lb[k, sl16] = jnp.where(a >= 0, a, 0.2 * a) + padoff

            @pl.loop(0, 8)
            def _(j):
                sl16 = pl.ds(j * 16, 16)
                kk, vv = plsc.sort_key_val(dstb[k, sl16], lb[k, sl16])
                kf = kk.astype(jnp.float32)
                shb[pl.ds(16, 16)] = kf
                shb[pl.ds(48, 16)] = vv
                for sh in (1, 2, 4, 8):
                    pk = shb[pl.ds(16 - sh, 16)]
                    pv = shb[pl.ds(48 - sh, 16)]
                    take = pk == kf
                    vv = jnp.where(take, jnp.maximum(vv, pv), vv)
                    if sh < 8:
                        shb[pl.ds(48, 16)] = vv
                nk = shb[pl.ds(17, 16)]
                islast = nk != kf
                cur = plsc.load_gather(m_t, [kk])
                plsc.store_scatter(m_t, [kk], jnp.maximum(cur, vv),
                                   mask=islast)

        pltpu.sync_copy(lb, l_hbm.at[pl.ds(rb, 8)])

    pltpu.sync_copy(m_t, mpart_hbm.at[wid])


def _run_pass1(srcp, dstp, ea2d, v3p, asrc1, adst1):
    kfn = pl.kernel(
        _sc_pass1,
        out_type=[
            jax.ShapeDtypeStruct((ER, 128), jnp.float32),
            jax.ShapeDtypeStruct((32, NP), jnp.float32),
        ],
        mesh=_MESH,
        scratch_types=[
            pltpu.MemorySpace.VMEM_SHARED((NP,), jnp.float32),
            pltpu.MemorySpace.VMEM_SHARED((NP,), jnp.float32),
            pltpu.MemorySpace.VMEM((NP,), jnp.float32),
            pltpu.MemorySpace.VMEM((8, 128), jnp.int32),
            pltpu.MemorySpace.VMEM((8, 128), jnp.int32),
            pltpu.MemorySpace.VMEM((24, 128), jnp.float32),
            pltpu.MemorySpace.VMEM((8, 128), jnp.float32),
            pltpu.MemorySpace.VMEM((8, 128), jnp.float32),
            pltpu.MemorySpace.VMEM((8, 128), jnp.float32),
            pltpu.MemorySpace.VMEM((64,), jnp.float32),
            pltpu.MemorySpace.VMEM((16,), jnp.float32),
            pltpu.SemaphoreType.DMA,
            pltpu.SemaphoreType.DMA,
        ],
        compiler_params=_SC_PARAMS,
    )
    return kfn(srcp, dstp, ea2d, v3p, asrc1, adst1)


def _sc_pass2(src_hbm, dst_hbm, l_hbm, m_hbm, h01_hbm, acc0_hbm, den0_hbm,
              num_hbm, den_hbm,
              acc_sh, den_sh, m_sh, srcb, dstb, lbuf, mvb, exb, rbig,
              gs0, gs1, gs2, gs3, vsem, as0, as1, as2, as3, dsem):
    gsems = (gs0, gs1, gs2, gs3)
    asems = (as0, as1, as2, as3)
    c = lax.axis_index("c")
    s = lax.axis_index("s")
    rows0 = s * (NP // 16)
    rowbase = s * ROWS_D

    @pl.when(s == 0)
    def _():
        pltpu.sync_copy(m_hbm, m_sh)

    @pl.when(s == 1)
    def _():
        pltpu.sync_copy(den0_hbm, den_sh)

    cNP = c * NP

    pltpu.sync_copy(acc0_hbm.at[pl.ds(cNP + rows0, NP // 16)],
                    acc_sh.at[pl.ds(rows0, NP // 16)])

    plsc.subcore_barrier()

    def main_loop(h_hbm, with_den):
        @pl.loop(0, ROWS_D // 8)
        def _(ch):
            rb = rowbase + ch * 8
            pltpu.sync_copy(src_hbm.at[pl.ds(rb, 8)], srcb)
            pltpu.sync_copy(dst_hbm.at[pl.ds(rb, 8)], dstb)
            pltpu.sync_copy(l_hbm.at[pl.ds(rb, 8)], lbuf)

            @pl.loop(0, 8)
            def _(k):
                @pl.loop(0, 8)
                def _(j):
                    sl16 = pl.ds(j * 16, 16)
                    srcb[k, sl16] = srcb[k, sl16] + cNP

            mv_d = [pltpu.async_copy(m_sh.at[dstb.at[k]], mvb.at[k], vsem)
                    for k in range(8)]
            for d in mv_d:
                d.wait()
            g_d = [None] * 8
            for k in range(3):
                g_d[k] = pltpu.async_copy(h_hbm.at[srcb.at[k]],
                                          rbig.at[k % 4], gsems[k % 4])
            sc_d = [None] * 8
            den_d = []
            for k in range(8):
                @pl.loop(0, 8)
                def _(j, k=k):
                    sl16 = pl.ds(j * 16, 16)
                    exb[k, sl16] = jnp.exp(lbuf[k, sl16] - mvb[k, sl16])

                g_d[k].wait()
                if k + 3 < 8:
                    if k >= 1:
                        sc_d[k - 1].wait()
                    g_d[k + 3] = pltpu.async_copy(
                        h_hbm.at[srcb.at[k + 3]], rbig.at[(k + 3) % 4],
                        gsems[(k + 3) % 4])

                @pl.loop(0, 8)
                def _(q, k=k):
                    exq = exb[k, pl.ds(q * 16, 16)]
                    for t in range(16):
                        jj = q * 16 + t
                        rbig[k % 4, jj, pl.ds(0, HH)] = (
                            rbig[k % 4, jj, pl.ds(0, HH)] * exq[t])

                sc_d[k] = pltpu.async_copy(rbig.at[k % 4],
                                           acc_sh.at[dstb.at[k]],
                                           asems[k % 4], add=True)

                if with_den:
                    den_d.append(pltpu.async_copy(
                        exb.at[k], den_sh.at[dstb.at[k]], dsem, add=True))

            for d in den_d:
                d.wait()
            sc_d[4].wait()
            sc_d[5].wait()
            sc_d[6].wait()
            sc_d[7].wait()

    @pl.when(c == 0)
    def _():
        main_loop(h01_hbm, True)

    @pl.when(c == 1)
    def _():
        main_loop(h01_hbm, False)

    plsc.subcore_barrier()

    pltpu.sync_copy(acc_sh.at[pl.ds(rows0, NP // 16)],
                    num_hbm.at[pl.ds(cNP + rows0, NP // 16)])

    @pl.when(jnp.logical_and(c == 0, s == 0))
    def _():
        pltpu.sync_copy(den_sh, den_hbm)


def _run_pass2(srcp, dstp, l2d, m1, h01, acc0, den01):
    kfn = pl.kernel(
        _sc_pass2,
        out_type=[
            jax.ShapeDtypeStruct((2 * NP, HH), jnp.float32),
            jax.ShapeDtypeStruct((NP,), jnp.float32),
        ],
        mesh=_MESH,
        scratch_types=[
            pltpu.MemorySpace.VMEM_SHARED((NP, HH), jnp.float32),
            pltpu.MemorySpace.VMEM_SHARED((NP,), jnp.float32),
            pltpu.MemorySpace.VMEM_SHARED((NP,), jnp.float32),
            pltpu.MemorySpace.VMEM((8, 128), jnp.int32),
            pltpu.MemorySpace.VMEM((8, 128), jnp.int32),
            pltpu.MemorySpace.VMEM((8, 128), jnp.float32),
            pltpu.MemorySpace.VMEM((8, 128), jnp.float32),
            pltpu.MemorySpace.VMEM((8, 128), jnp.float32),
            pltpu.MemorySpace.VMEM((4, 128, HH), jnp.float32),
            pltpu.SemaphoreType.DMA,
            pltpu.SemaphoreType.DMA,
            pltpu.SemaphoreType.DMA,
            pltpu.SemaphoreType.DMA,
            pltpu.SemaphoreType.DMA,
            pltpu.SemaphoreType.DMA,
            pltpu.SemaphoreType.DMA,
            pltpu.SemaphoreType.DMA,
            pltpu.SemaphoreType.DMA,
            pltpu.SemaphoreType.DMA,
        ],
        compiler_params=_SC_PARAMS,
    )
    return kfn(srcp, dstp, l2d, m1, h01, acc0, den01)


@jax.jit
def kernel(x, edge_index, edge_attr, batch, W, att_src, att_dst, W_edge,
           att_edge, bias_conv, att_lin_W, att_lin_b, out_W, out_b):
    v3 = W_edge @ att_edge
    eaf = edge_attr.reshape(E * 3)
    ea2d = eaf.reshape(EA_ROWS, 128)
    s384 = jnp.sum(eaf.reshape(ER_REAL, 384), axis=0)
    c = ((jnp.sum(s384[0::3]) * v3[0] + jnp.sum(s384[1::3]) * v3[1]
          + jnp.sum(s384[2::3]) * v3[2]) / E).reshape(1, 1)
    v3p = jnp.pad(v3, (0, 13))

    xp = jnp.pad(x, ((0, NP - N), (0, 0)))
    x8 = xp.reshape(NP // 8, 8 * F_IN)
    gidx = jnp.arange(128) // HH
    sel8 = (gidx[None, :] == jnp.arange(8)[:, None]).astype(jnp.float32)
    wcol = jnp.arange(128) % HH
    W0big = jnp.where(
        (jnp.arange(8 * F_IN)[:, None] // F_IN) == gidx[None, :],
        W[jnp.arange(8 * F_IN)[:, None] % F_IN, wcol[None, :]], 0.0)
    W1big = jnp.where(
        (jnp.arange(8 * F_IN)[:, None] // F_IN) == gidx[None, :],
        W[jnp.arange(8 * F_IN)[:, None] % F_IN, wcol[None, :] + HH], 0.0)
    batchp = jnp.pad(batch, (0, NP - N), constant_values=G)
    h0, h1, asrc, adst, sl = _node_proj(xp, x8, W, W0big, W1big,
                                        att_src, att_dst, c)

    pad = EPAD - E
    srcp = jnp.pad(edge_index[0], (0, pad)).reshape(ER, 128)
    dstp = jnp.pad(edge_index[1], (0, pad)).reshape(ER, 128)

    l2d, mpart = _run_pass1(srcp, dstp, ea2d, v3p, asrc, adst)

    m, den0 = _combine_max(mpart, sl)
    a00, a01 = _scale_acc0(h0, h1, den0.reshape(NP // 8, 8), sel8)

    h01 = jnp.concatenate([h0, h1], axis=0).reshape(2 * NP, HH)
    acc0 = jnp.concatenate([a00, a01], axis=0).reshape(2 * NP, HH)
    num, den = _run_pass2(srcp, dstp, l2d, m, h01, acc0, den0)
    num0 = num[:NP]
    num1 = num[NP:]

    bias2 = bias_conv.reshape(1, H)
    gate, mg = _node_gate(num0, num1, den, bias2,
                          att_lin_W.reshape(1, H),
                          att_lin_b.reshape(1, 1), batchp)

    _, _, y = _pool(num0, num1, den, bias2, gate, batchp, mg,
                    out_W, out_b.reshape(1, 1))
    return y
